# baseline (speedup 1.0000x reference)
import jax
import jax.numpy as jnp
from jax import lax
from jax.experimental import pallas as pl
from jax.experimental.pallas import tpu as pltpu
from jax.experimental.pallas import tpu_sc as plsc

N = 10000
E = 320000
DI = 128
DH = 256
NC = 2
NS = 16
LANES = 16
DEGW = 8
HD = DI // NC
CHUNK = 128
CPW = 160
CPWD = CPW // NC
E_PAD = NS * CPW * CHUNK
N_PAD = 10112
RPS = N_PAD // NS

_MESH = dict(core_axis_name="c", subcore_axis_name="s")


def _zero_stripe(zb, sh, ro):
  off = 0
  for sz in (CHUNK, CHUNK, CHUNK, CHUNK, RPS - 4 * CHUNK):
    pltpu.sync_copy(zb.at[pl.ds(0, sz)], sh.at[pl.ds(ro + off, sz)])
    off += sz


def _seg_sum_body(with_deg, refs):
  if with_deg:
    (table, srcw, dstw, ones8, zeros8, part, degp, src_v, dst_v,
     rows0, rows1, rows2, rows3, zb, ones_v, zb8, acc_sh, deg_sh,
     gs0, gs1, gs2, gs3, ss0, ss1, ss2, ss3, dsem) = refs
  else:
    (table, srcw, dstw, part, src_v, dst_v,
     rows0, rows1, rows2, rows3, zb, acc_sh,
     gs0, gs1, gs2, gs3, ss0, ss1, ss2, ss3) = refs
  rows = (rows0, rows1, rows2, rows3)
  gsem = (gs0, gs1, gs2, gs3)
  ssem = (ss0, ss1, ss2, ss3)

  c = lax.axis_index("c")
  s = lax.axis_index("s")
  ro = s * RPS

  def zrow(i, carry):
    for j in range(HD // LANES):
      zb[i, 16 * j:16 * j + 16] = jnp.zeros((LANES,), jnp.float32)
    return carry
  lax.fori_loop(0, CHUNK, zrow, 0)

  _zero_stripe(zb, acc_sh, ro)
  if with_deg:
    pltpu.sync_copy(ones8, ones_v)
    pltpu.sync_copy(zeros8, zb8)
    _zero_stripe(zb8, deg_sh, ro)

  pltpu.sync_copy(srcw.at[s], src_v)
  pltpu.sync_copy(dstw.at[s], dst_v)

  plsc.subcore_barrier()

  def gather(ci, b):
    pltpu.async_copy(table.at[c].at[src_v.at[ci]], rows[b], gsem[b])

  def gather_wait(b):
    pltpu.make_async_copy(table.at[c].at[src_v.at[0]], rows[b], gsem[b]).wait()

  def scatter(ci, b):
    pltpu.async_copy(rows[b], acc_sh.at[dst_v.at[ci]], ssem[b], add=True)

  def scatter_wait(b):
    pltpu.make_async_copy(rows[b], acc_sh.at[dst_v.at[0]], ssem[b]).wait()

  def deg_scatter(ci):
    pltpu.async_copy(ones_v, deg_sh.at[dst_v.at[ci]], dsem, add=True)

  def deg_wait():
    pltpu.make_async_copy(ones_v, deg_sh.at[dst_v.at[0]], dsem).wait()

  gather(0, 0)
  gather(1, 1)

  def step(k, carry):
    for b in range(4):
      ci = 4 * k + b

      @pl.when(ci >= 2)
      def _(b=b):
        scatter_wait((b + 2) % 4)

      @pl.when(ci + 2 < CPW)
      def _(b=b, ci=ci):
        gather(ci + 2, (b + 2) % 4)

      gather_wait(b)
      scatter(ci, b)

      if with_deg:
        deg_scatter(ci)

        @pl.when(ci >= 4)
        def _():
          deg_wait()
    return carry
  lax.fori_loop(0, CPW // 4, step, 0)

  scatter_wait(2)
  scatter_wait(3)
  if with_deg:
    for _ in range(4):
      deg_wait()

  plsc.subcore_barrier()

  pltpu.sync_copy(acc_sh.at[pl.ds(ro, RPS)], part.at[c, pl.ds(ro, RPS)])
  if with_deg:
    pltpu.sync_copy(deg_sh.at[pl.ds(ro, RPS)], degp.at[c, pl.ds(ro, RPS)])


def _make_seg_sum(with_deg):
  scratch = [
      pltpu.VMEM((CPW, CHUNK), jnp.int32),
      pltpu.VMEM((CPW, CHUNK), jnp.int32),
      pltpu.VMEM((CHUNK, HD), jnp.float32),
      pltpu.VMEM((CHUNK, HD), jnp.float32),
      pltpu.VMEM((CHUNK, HD), jnp.float32),
      pltpu.VMEM((CHUNK, HD), jnp.float32),
      pltpu.VMEM((CHUNK, HD), jnp.float32),
  ]
  if with_deg:
    scratch += [
        pltpu.VMEM((CHUNK, DEGW), jnp.float32),
        pltpu.VMEM((CHUNK, DEGW), jnp.float32),
    ]
  scratch += [pltpu.VMEM_SHARED((N_PAD, HD), jnp.float32)]
  if with_deg:
    scratch += [pltpu.VMEM_SHARED((N_PAD, DEGW), jnp.float32)]
  scratch += [pltpu.SemaphoreType.DMA] * 8
  if with_deg:
    scratch += [pltpu.SemaphoreType.DMA]
  out_type = [jax.ShapeDtypeStruct((NC, N_PAD, HD), jnp.float32)]
  if with_deg:
    out_type += [jax.ShapeDtypeStruct((NC, N_PAD, DEGW), jnp.float32)]
  return pl.kernel(
      lambda *refs: _seg_sum_body(with_deg, refs),
      out_type=tuple(out_type),
      mesh=plsc.VectorSubcoreMesh(**_MESH),
      scratch_types=tuple(scratch),
      compiler_params=pltpu.CompilerParams(use_tc_tiling_on_sc=False),
      name="seg_sum_deg" if with_deg else "seg_sum",
  )


_seg_sum = _make_seg_sum(with_deg=False)
_seg_sum_deg = _make_seg_sum(with_deg=True)

_DN = (((1,), (1,)), ((), ()))
_RB = 1000


def _deg_inv(degp_ref, i):
  deg = (degp_ref[0, pl.ds(i * _RB, _RB), 0:1]
         + degp_ref[1, pl.ds(i * _RB, _RB), 0:1])
  return 2.0 / jnp.maximum(deg, 2.0)


def _tc1_body(part_ref, degp_ref, x_ref, wl1_ref, bl1_ref, wr1_ref,
              wl2_ref, wr2_ref, bl2_ref, p_ref, q_ref):
  i = pl.program_id(0)
  inv = _deg_inv(degp_ref, i)
  acc = jnp.concatenate([part_ref[0], part_ref[1]], axis=1)
  mean = acc * inv
  h = (lax.dot_general(mean, wl1_ref[...], _DN, preferred_element_type=jnp.float32)
       + bl1_ref[...]
       + lax.dot_general(x_ref[...], wr1_ref[...], _DN,
                         preferred_element_type=jnp.float32))
  p = lax.dot_general(h, wl2_ref[...], _DN, preferred_element_type=jnp.float32)
  p_ref[0] = p[:, :HD]
  p_ref[1] = p[:, HD:]
  q_ref[...] = (lax.dot_general(h, wr2_ref[...], _DN,
                                preferred_element_type=jnp.float32)
                + bl2_ref[...])


def _tc2_body(part_ref, degp_ref, q_ref, out_ref):
  i = pl.program_id(0)
  inv = _deg_inv(degp_ref, i)
  acc = jnp.concatenate([part_ref[0], part_ref[1]], axis=1)
  out_ref[...] = acc * inv + q_ref[...]


def _tc1(part, degp, x, Wl1, bl1, Wr1, Wl2, Wr2, bl2):
  grid = (N // _RB,)
  return pl.pallas_call(
      _tc1_body,
      grid=grid,
      in_specs=[
          pl.BlockSpec((NC, _RB, HD), lambda i: (0, i, 0)),
          pl.BlockSpec((NC, N_PAD, DEGW), lambda i: (0, 0, 0)),
          pl.BlockSpec((_RB, DI), lambda i: (i, 0)),
          pl.BlockSpec((DH, DI), lambda i: (0, 0)),
          pl.BlockSpec((1, DH), lambda i: (0, 0)),
          pl.BlockSpec((DH, DI), lambda i: (0, 0)),
          pl.BlockSpec((DI, DH), lambda i: (0, 0)),
          pl.BlockSpec((DI, DH), lambda i: (0, 0)),
          pl.BlockSpec((1, DI), lambda i: (0, 0)),
      ],
      out_specs=[
          pl.BlockSpec((NC, _RB, HD), lambda i: (0, i, 0)),
          pl.BlockSpec((_RB, DI), lambda i: (i, 0)),
      ],
      out_shape=[
          jax.ShapeDtypeStruct((NC, N, HD), jnp.float32),
          jax.ShapeDtypeStruct((N, DI), jnp.float32),
      ],
      name="sage_dense1",
  )(part, degp, x, Wl1, bl1, Wr1, Wl2, Wr2, bl2)


def _tc2(part, degp, q):
  grid = (N // _RB,)
  return pl.pallas_call(
      _tc2_body,
      grid=grid,
      in_specs=[
          pl.BlockSpec((NC, _RB, HD), lambda i: (0, i, 0)),
          pl.BlockSpec((NC, N_PAD, DEGW), lambda i: (0, 0, 0)),
          pl.BlockSpec((_RB, DI), lambda i: (i, 0)),
      ],
      out_specs=pl.BlockSpec((_RB, DI), lambda i: (i, 0)),
      out_shape=jax.ShapeDtypeStruct((N, DI), jnp.float32),
      name="sage_dense2",
  )(part, degp, q)


def kernel(x, edge_index, Wl1, bl1, Wr1, Wl2, bl2, Wr2):
  src = edge_index[0]
  dst = edge_index[1]
  pad = E_PAD - E
  pad_src = jnp.arange(pad, dtype=jnp.int32) % N
  pad_dst = N + jnp.arange(pad, dtype=jnp.int32) % (N_PAD - N)
  srcw = jnp.concatenate([src, pad_src]).reshape(NS, CPW, CHUNK)
  dstw = jnp.concatenate([dst, pad_dst]).reshape(NS, CPW, CHUNK)
  xh = jnp.stack([x[:, :HD], x[:, HD:]], axis=0)

  part1, degp = _seg_sum_deg(xh, srcw, dstw,
                             jnp.ones((CHUNK, DEGW), jnp.float32),
                             jnp.zeros((CHUNK, DEGW), jnp.float32))
  p, q = _tc1(part1, degp, x, Wl1, bl1[None, :], Wr1, Wl2, Wr2, bl2[None, :])
  (part2,) = _seg_sum(p, srcw, dstw)
  return _tc2(part2, degp, q)


# speedup vs baseline: 1.0394x; 1.0394x over previous
import jax
import jax.numpy as jnp
from jax import lax
from jax.experimental import pallas as pl
from jax.experimental.pallas import tpu as pltpu
from jax.experimental.pallas import tpu_sc as plsc

N = 10000
E = 320000
DI = 128
DH = 256
NC = 2
NS = 16
LANES = 16
HD = DI // NC
CHUNK = 128
CPW = 160
CPWD = CPW // NC
E_PAD = NS * CPW * CHUNK
N_PAD = 10112
RPS = N_PAD // NS

_MESH = dict(core_axis_name="c", subcore_axis_name="s")


def _zero_stripe(zb, sh, ro):
  off = 0
  for sz in (CHUNK, CHUNK, CHUNK, CHUNK, RPS - 4 * CHUNK):
    pltpu.sync_copy(zb.at[pl.ds(0, sz)], sh.at[pl.ds(ro + off, sz)])
    off += sz


def _seg_sum_body(table, srcw, dstw, part, src_v, dst_v,
                  rows0, rows1, rows2, rows3, zb, acc_sh,
                  gs0, gs1, gs2, gs3, ss0, ss1, ss2, ss3):
  rows = (rows0, rows1, rows2, rows3)
  gsem = (gs0, gs1, gs2, gs3)
  ssem = (ss0, ss1, ss2, ss3)

  c = lax.axis_index("c")
  s = lax.axis_index("s")
  ro = s * RPS

  def zrow(i, carry):
    for j in range(HD // LANES):
      zb[i, 16 * j:16 * j + 16] = jnp.zeros((LANES,), jnp.float32)
    return carry
  lax.fori_loop(0, CHUNK, zrow, 0)

  _zero_stripe(zb, acc_sh, ro)

  pltpu.sync_copy(srcw.at[s], src_v)
  pltpu.sync_copy(dstw.at[s], dst_v)

  plsc.subcore_barrier()

  def gather(ci, b):
    pltpu.async_copy(table.at[c].at[src_v.at[ci]], rows[b], gsem[b])

  def gather_wait(b):
    pltpu.make_async_copy(table.at[c].at[src_v.at[0]], rows[b], gsem[b]).wait()

  def scatter(ci, b):
    pltpu.async_copy(rows[b], acc_sh.at[dst_v.at[ci]], ssem[b], add=True)

  def scatter_wait(b):
    pltpu.make_async_copy(rows[b], acc_sh.at[dst_v.at[0]], ssem[b]).wait()

  gather(0, 0)
  gather(1, 1)

  def step(k, carry):
    for b in range(4):
      ci = 4 * k + b

      @pl.when(ci >= 2)
      def _(b=b):
        scatter_wait((b + 2) % 4)

      @pl.when(ci + 2 < CPW)
      def _(b=b, ci=ci):
        gather(ci + 2, (b + 2) % 4)

      gather_wait(b)
      scatter(ci, b)
    return carry
  lax.fori_loop(0, CPW // 4, step, 0)

  scatter_wait(2)
  scatter_wait(3)

  plsc.subcore_barrier()

  pltpu.sync_copy(acc_sh.at[pl.ds(ro, RPS)], part.at[c, pl.ds(ro, RPS)])


def _make_seg_sum():
  scratch = (
      pltpu.VMEM((CPW, CHUNK), jnp.int32),
      pltpu.VMEM((CPW, CHUNK), jnp.int32),
      pltpu.VMEM((CHUNK, HD), jnp.float32),
      pltpu.VMEM((CHUNK, HD), jnp.float32),
      pltpu.VMEM((CHUNK, HD), jnp.float32),
      pltpu.VMEM((CHUNK, HD), jnp.float32),
      pltpu.VMEM((CHUNK, HD), jnp.float32),
      pltpu.VMEM_SHARED((N_PAD, HD), jnp.float32),
      pltpu.SemaphoreType.DMA,
      pltpu.SemaphoreType.DMA,
      pltpu.SemaphoreType.DMA,
      pltpu.SemaphoreType.DMA,
      pltpu.SemaphoreType.DMA,
      pltpu.SemaphoreType.DMA,
      pltpu.SemaphoreType.DMA,
      pltpu.SemaphoreType.DMA,
  )
  return pl.kernel(
      _seg_sum_body,
      out_type=(jax.ShapeDtypeStruct((NC, N_PAD, HD), jnp.float32),),
      mesh=plsc.VectorSubcoreMesh(**_MESH),
      scratch_types=scratch,
      compiler_params=pltpu.CompilerParams(use_tc_tiling_on_sc=False),
      name="seg_sum",
  )


def _deg_body(dstw, degp, dst_v, ones_v, zb16, deg_sh, dsem):
  c = lax.axis_index("c")
  s = lax.axis_index("s")
  ro = s * RPS

  def frow(i, carry):
    ones_v[i] = jnp.ones((LANES,), jnp.float32)
    zb16[i] = jnp.zeros((LANES,), jnp.float32)
    return carry
  lax.fori_loop(0, CHUNK, frow, 0)

  _zero_stripe(zb16, deg_sh, ro)

  pltpu.sync_copy(dstw.at[s, pl.ds(c * CPWD, CPWD)], dst_v)

  plsc.subcore_barrier()

  def deg_wait():
    pltpu.make_async_copy(ones_v, deg_sh.at[dst_v.at[0]], dsem).wait()

  def step(k, carry):
    pltpu.async_copy(ones_v, deg_sh.at[dst_v.at[k]], dsem, add=True)

    @pl.when(k >= 4)
    def _():
      deg_wait()
    return carry
  lax.fori_loop(0, CPWD, step, 0)

  for _ in range(4):
    deg_wait()

  plsc.subcore_barrier()

  pltpu.sync_copy(deg_sh.at[pl.ds(ro, RPS)], degp.at[c, pl.ds(ro, RPS)])


def _make_deg():
  scratch = (
      pltpu.VMEM((CPWD, CHUNK), jnp.int32),
      pltpu.VMEM((CHUNK, LANES), jnp.float32),
      pltpu.VMEM((CHUNK, LANES), jnp.float32),
      pltpu.VMEM_SHARED((N_PAD, LANES), jnp.float32),
      pltpu.SemaphoreType.DMA,
  )
  return pl.kernel(
      _deg_body,
      out_type=(jax.ShapeDtypeStruct((NC, N_PAD, LANES), jnp.float32),),
      mesh=plsc.VectorSubcoreMesh(**_MESH),
      scratch_types=scratch,
      compiler_params=pltpu.CompilerParams(use_tc_tiling_on_sc=False),
      name="deg_count",
  )


_seg_sum = _make_seg_sum()
_deg_count = _make_deg()

_DN = (((1,), (1,)), ((), ()))
_RB = 1000


def _deg_inv(degp_ref, i):
  deg = (degp_ref[0, pl.ds(i * _RB, _RB), 0:1]
         + degp_ref[1, pl.ds(i * _RB, _RB), 0:1])
  return 1.0 / jnp.maximum(deg, 1.0)


def _tc1_body(part_ref, degp_ref, x_ref, wl1_ref, bl1_ref, wr1_ref,
              wl2_ref, wr2_ref, bl2_ref, p_ref, q_ref):
  i = pl.program_id(0)
  inv = _deg_inv(degp_ref, i)
  acc = jnp.concatenate([part_ref[0], part_ref[1]], axis=1)
  mean = acc * inv
  h = (lax.dot_general(mean, wl1_ref[...], _DN, preferred_element_type=jnp.float32)
       + bl1_ref[...]
       + lax.dot_general(x_ref[...], wr1_ref[...], _DN,
                         preferred_element_type=jnp.float32))
  p = lax.dot_general(h, wl2_ref[...], _DN, preferred_element_type=jnp.float32)
  p_ref[0] = p[:, :HD]
  p_ref[1] = p[:, HD:]
  q_ref[...] = (lax.dot_general(h, wr2_ref[...], _DN,
                                preferred_element_type=jnp.float32)
                + bl2_ref[...])


def _tc2_body(part_ref, degp_ref, q_ref, out_ref):
  i = pl.program_id(0)
  inv = _deg_inv(degp_ref, i)
  acc = jnp.concatenate([part_ref[0], part_ref[1]], axis=1)
  out_ref[...] = acc * inv + q_ref[...]


def _tc1(part, degp, x, Wl1, bl1, Wr1, Wl2, Wr2, bl2):
  grid = (N // _RB,)
  return pl.pallas_call(
      _tc1_body,
      grid=grid,
      in_specs=[
          pl.BlockSpec((NC, _RB, HD), lambda i: (0, i, 0)),
          pl.BlockSpec((NC, N_PAD, LANES), lambda i: (0, 0, 0)),
          pl.BlockSpec((_RB, DI), lambda i: (i, 0)),
          pl.BlockSpec((DH, DI), lambda i: (0, 0)),
          pl.BlockSpec((1, DH), lambda i: (0, 0)),
          pl.BlockSpec((DH, DI), lambda i: (0, 0)),
          pl.BlockSpec((DI, DH), lambda i: (0, 0)),
          pl.BlockSpec((DI, DH), lambda i: (0, 0)),
          pl.BlockSpec((1, DI), lambda i: (0, 0)),
      ],
      out_specs=[
          pl.BlockSpec((NC, _RB, HD), lambda i: (0, i, 0)),
          pl.BlockSpec((_RB, DI), lambda i: (i, 0)),
      ],
      out_shape=[
          jax.ShapeDtypeStruct((NC, N, HD), jnp.float32),
          jax.ShapeDtypeStruct((N, DI), jnp.float32),
      ],
      name="sage_dense1",
  )(part, degp, x, Wl1, bl1, Wr1, Wl2, Wr2, bl2)


def _tc2(part, degp, q):
  grid = (N // _RB,)
  return pl.pallas_call(
      _tc2_body,
      grid=grid,
      in_specs=[
          pl.BlockSpec((NC, _RB, HD), lambda i: (0, i, 0)),
          pl.BlockSpec((NC, N_PAD, LANES), lambda i: (0, 0, 0)),
          pl.BlockSpec((_RB, DI), lambda i: (i, 0)),
      ],
      out_specs=pl.BlockSpec((_RB, DI), lambda i: (i, 0)),
      out_shape=jax.ShapeDtypeStruct((N, DI), jnp.float32),
      name="sage_dense2",
  )(part, degp, q)


def kernel(x, edge_index, Wl1, bl1, Wr1, Wl2, bl2, Wr2):
  src = edge_index[0]
  dst = edge_index[1]
  pad = E_PAD - E
  pad_src = jnp.arange(pad, dtype=jnp.int32) % N
  pad_dst = N + jnp.arange(pad, dtype=jnp.int32) % (N_PAD - N)
  srcw = jnp.concatenate([src, pad_src]).reshape(NS, CPW, CHUNK)
  dstw = jnp.concatenate([dst, pad_dst]).reshape(NS, CPW, CHUNK)
  xh = jnp.stack([x[:, :HD], x[:, HD:]], axis=0)

  (part1,) = _seg_sum(xh, srcw, dstw)
  (degp,) = _deg_count(dstw)
  p, q = _tc1(part1, degp, x, Wl1, bl1[None, :], Wr1, Wl2, Wr2, bl2[None, :])
  (part2,) = _seg_sum(p, srcw, dstw)
  return _tc2(part2, degp, q)


# speedup vs baseline: 1.0493x; 1.0095x over previous
import jax
import jax.numpy as jnp
from jax import lax
from jax.experimental import pallas as pl
from jax.experimental.pallas import tpu as pltpu
from jax.experimental.pallas import tpu_sc as plsc

N = 10000
E = 320000
DI = 128
DH = 256
NC = 2
NS = 16
LANES = 16
HD = DI // NC
CHUNK = 128
EPS = E // NS
CPW = EPS // CHUNK
TAIL = EPS - CPW * CHUNK
EPSD = EPS // NC
CPWD = EPSD // CHUNK
TAILD = EPSD - CPWD * CHUNK
N_PAD = 10112
RPS = N_PAD // NS

_MESH = dict(core_axis_name="c", subcore_axis_name="s")


def _zero_stripe(zb, sh, ro):
  off = 0
  for sz in (CHUNK, CHUNK, CHUNK, CHUNK, RPS - 4 * CHUNK):
    pltpu.sync_copy(zb.at[pl.ds(0, sz)], sh.at[pl.ds(ro + off, sz)])
    off += sz


def _seg_sum_body(table, srcw, dstw, part, src_v, dst_v,
                  rows0, rows1, rows2, rows3, zb, acc_sh,
                  gs0, gs1, gs2, gs3, ss0, ss1, ss2, ss3):
  rows = (rows0, rows1, rows2, rows3)
  gsem = (gs0, gs1, gs2, gs3)
  ssem = (ss0, ss1, ss2, ss3)

  c = lax.axis_index("c")
  s = lax.axis_index("s")
  ro = s * RPS

  def zrow(i, carry):
    for j in range(HD // LANES):
      zb[i, 16 * j:16 * j + 16] = jnp.zeros((LANES,), jnp.float32)
    return carry
  lax.fori_loop(0, CHUNK, zrow, 0)

  _zero_stripe(zb, acc_sh, ro)

  pltpu.sync_copy(srcw.at[pl.ds(s * EPS, EPS)], src_v)
  pltpu.sync_copy(dstw.at[pl.ds(s * EPS, EPS)], dst_v)

  plsc.subcore_barrier()

  def gather(ci, b):
    pltpu.async_copy(table.at[c].at[src_v.at[pl.ds(ci * CHUNK, CHUNK)]],
                     rows[b], gsem[b])

  def gather_wait(b):
    pltpu.make_async_copy(table.at[c].at[src_v.at[pl.ds(0, CHUNK)]],
                          rows[b], gsem[b]).wait()

  def scatter(ci, b):
    pltpu.async_copy(rows[b], acc_sh.at[dst_v.at[pl.ds(ci * CHUNK, CHUNK)]],
                     ssem[b], add=True)

  def scatter_wait(b):
    pltpu.make_async_copy(rows[b], acc_sh.at[dst_v.at[pl.ds(0, CHUNK)]],
                          ssem[b]).wait()

  gather(0, 0)
  gather(1, 1)

  def step(k, carry):
    for b in range(4):
      ci = 4 * k + b

      @pl.when(ci >= 2)
      def _(b=b):
        scatter_wait((b + 2) % 4)

      @pl.when(ci + 2 < CPW)
      def _(b=b, ci=ci):
        gather(ci + 2, (b + 2) % 4)

      gather_wait(b)
      scatter(ci, b)
    return carry
  lax.fori_loop(0, CPW // 4, step, 0)

  scatter_wait(2)
  scatter_wait(3)

  tsrc = table.at[c].at[src_v.at[pl.ds(CPW * CHUNK, TAIL)]]
  trow = rows0.at[pl.ds(0, TAIL)]
  tdst = acc_sh.at[dst_v.at[pl.ds(CPW * CHUNK, TAIL)]]
  pltpu.async_copy(tsrc, trow, gs0)
  pltpu.make_async_copy(tsrc, trow, gs0).wait()
  pltpu.async_copy(trow, tdst, ss0, add=True)
  pltpu.make_async_copy(trow, tdst, ss0).wait()

  plsc.subcore_barrier()

  pltpu.sync_copy(acc_sh.at[pl.ds(ro, RPS)], part.at[c, pl.ds(ro, RPS)])


def _make_seg_sum():
  scratch = (
      pltpu.VMEM((EPS,), jnp.int32),
      pltpu.VMEM((EPS,), jnp.int32),
      pltpu.VMEM((CHUNK, HD), jnp.float32),
      pltpu.VMEM((CHUNK, HD), jnp.float32),
      pltpu.VMEM((CHUNK, HD), jnp.float32),
      pltpu.VMEM((CHUNK, HD), jnp.float32),
      pltpu.VMEM((CHUNK, HD), jnp.float32),
      pltpu.VMEM_SHARED((N_PAD, HD), jnp.float32),
      pltpu.SemaphoreType.DMA,
      pltpu.SemaphoreType.DMA,
      pltpu.SemaphoreType.DMA,
      pltpu.SemaphoreType.DMA,
      pltpu.SemaphoreType.DMA,
      pltpu.SemaphoreType.DMA,
      pltpu.SemaphoreType.DMA,
      pltpu.SemaphoreType.DMA,
  )
  return pl.kernel(
      _seg_sum_body,
      out_type=(jax.ShapeDtypeStruct((NC, N_PAD, HD), jnp.float32),),
      mesh=plsc.VectorSubcoreMesh(**_MESH),
      scratch_types=scratch,
      compiler_params=pltpu.CompilerParams(use_tc_tiling_on_sc=False),
      name="seg_sum",
  )


def _deg_body(dstw, degp, dst_v, ones_v, zb16, deg_sh, dsem):
  c = lax.axis_index("c")
  s = lax.axis_index("s")
  ro = s * RPS

  def frow(i, carry):
    ones_v[i] = jnp.ones((LANES,), jnp.float32)
    zb16[i] = jnp.zeros((LANES,), jnp.float32)
    return carry
  lax.fori_loop(0, CHUNK, frow, 0)

  _zero_stripe(zb16, deg_sh, ro)

  pltpu.sync_copy(dstw.at[pl.ds(s * EPS + c * EPSD, EPSD)], dst_v)

  plsc.subcore_barrier()

  def deg_wait():
    pltpu.make_async_copy(ones_v, deg_sh.at[dst_v.at[pl.ds(0, CHUNK)]],
                          dsem).wait()

  def step(k, carry):
    pltpu.async_copy(ones_v, deg_sh.at[dst_v.at[pl.ds(k * CHUNK, CHUNK)]],
                     dsem, add=True)

    @pl.when(k >= 4)
    def _():
      deg_wait()
    return carry
  lax.fori_loop(0, CPWD, step, 0)

  for _ in range(4):
    deg_wait()

  tones = ones_v.at[pl.ds(0, TAILD)]
  tdst = deg_sh.at[dst_v.at[pl.ds(CPWD * CHUNK, TAILD)]]
  pltpu.async_copy(tones, tdst, dsem, add=True)
  pltpu.make_async_copy(tones, tdst, dsem).wait()

  plsc.subcore_barrier()

  pltpu.sync_copy(deg_sh.at[pl.ds(ro, RPS)], degp.at[c, pl.ds(ro, RPS)])


def _make_deg():
  scratch = (
      pltpu.VMEM((EPSD,), jnp.int32),
      pltpu.VMEM((CHUNK, LANES), jnp.float32),
      pltpu.VMEM((CHUNK, LANES), jnp.float32),
      pltpu.VMEM_SHARED((N_PAD, LANES), jnp.float32),
      pltpu.SemaphoreType.DMA,
  )
  return pl.kernel(
      _deg_body,
      out_type=(jax.ShapeDtypeStruct((NC, N_PAD, LANES), jnp.float32),),
      mesh=plsc.VectorSubcoreMesh(**_MESH),
      scratch_types=scratch,
      compiler_params=pltpu.CompilerParams(use_tc_tiling_on_sc=False),
      name="deg_count",
  )


_seg_sum = _make_seg_sum()
_deg_count = _make_deg()

_DN = (((1,), (1,)), ((), ()))
_RB = 1000


def _deg_inv(degp_ref, i):
  deg = (degp_ref[0, pl.ds(i * _RB, _RB), 0:1]
         + degp_ref[1, pl.ds(i * _RB, _RB), 0:1])
  return 1.0 / jnp.maximum(deg, 1.0)


def _tc1_body(part_ref, degp_ref, x_ref, wl1_ref, bl1_ref, wr1_ref,
              wl2_ref, wr2_ref, bl2_ref, p_ref, q_ref):
  i = pl.program_id(0)
  inv = _deg_inv(degp_ref, i)
  acc = jnp.concatenate([part_ref[0], part_ref[1]], axis=1)
  mean = acc * inv
  h = (lax.dot_general(mean, wl1_ref[...], _DN, preferred_element_type=jnp.float32)
       + bl1_ref[...]
       + lax.dot_general(x_ref[...], wr1_ref[...], _DN,
                         preferred_element_type=jnp.float32))
  p = lax.dot_general(h, wl2_ref[...], _DN, preferred_element_type=jnp.float32)
  p_ref[0] = p[:, :HD]
  p_ref[1] = p[:, HD:]
  q_ref[...] = (lax.dot_general(h, wr2_ref[...], _DN,
                                preferred_element_type=jnp.float32)
                + bl2_ref[...])


def _tc2_body(part_ref, degp_ref, q_ref, out_ref):
  i = pl.program_id(0)
  inv = _deg_inv(degp_ref, i)
  acc = jnp.concatenate([part_ref[0], part_ref[1]], axis=1)
  out_ref[...] = acc * inv + q_ref[...]


def _tc1(part, degp, x, Wl1, bl1, Wr1, Wl2, Wr2, bl2):
  grid = (N // _RB,)
  return pl.pallas_call(
      _tc1_body,
      grid=grid,
      in_specs=[
          pl.BlockSpec((NC, _RB, HD), lambda i: (0, i, 0)),
          pl.BlockSpec((NC, N_PAD, LANES), lambda i: (0, 0, 0)),
          pl.BlockSpec((_RB, DI), lambda i: (i, 0)),
          pl.BlockSpec((DH, DI), lambda i: (0, 0)),
          pl.BlockSpec((1, DH), lambda i: (0, 0)),
          pl.BlockSpec((DH, DI), lambda i: (0, 0)),
          pl.BlockSpec((DI, DH), lambda i: (0, 0)),
          pl.BlockSpec((DI, DH), lambda i: (0, 0)),
          pl.BlockSpec((1, DI), lambda i: (0, 0)),
      ],
      out_specs=[
          pl.BlockSpec((NC, _RB, HD), lambda i: (0, i, 0)),
          pl.BlockSpec((_RB, DI), lambda i: (i, 0)),
      ],
      out_shape=[
          jax.ShapeDtypeStruct((NC, N, HD), jnp.float32),
          jax.ShapeDtypeStruct((N, DI), jnp.float32),
      ],
      name="sage_dense1",
  )(part, degp, x, Wl1, bl1, Wr1, Wl2, Wr2, bl2)


def _tc2(part, degp, q):
  grid = (N // _RB,)
  return pl.pallas_call(
      _tc2_body,
      grid=grid,
      in_specs=[
          pl.BlockSpec((NC, _RB, HD), lambda i: (0, i, 0)),
          pl.BlockSpec((NC, N_PAD, LANES), lambda i: (0, 0, 0)),
          pl.BlockSpec((_RB, DI), lambda i: (i, 0)),
      ],
      out_specs=pl.BlockSpec((_RB, DI), lambda i: (i, 0)),
      out_shape=jax.ShapeDtypeStruct((N, DI), jnp.float32),
      name="sage_dense2",
  )(part, degp, q)


def kernel(x, edge_index, Wl1, bl1, Wr1, Wl2, bl2, Wr2):
  src = edge_index[0]
  dst = edge_index[1]
  xh = jnp.stack([x[:, :HD], x[:, HD:]], axis=0)

  (part1,) = _seg_sum(xh, src, dst)
  (degp,) = _deg_count(dst)
  p, q = _tc1(part1, degp, x, Wl1, bl1[None, :], Wr1, Wl2, Wr2, bl2[None, :])
  (part2,) = _seg_sum(p, src, dst)
  return _tc2(part2, degp, q)


# speedup vs baseline: 1.1219x; 1.0692x over previous
import jax
import jax.numpy as jnp
from jax import lax
from jax.experimental import pallas as pl
from jax.experimental.pallas import tpu as pltpu
from jax.experimental.pallas import tpu_sc as plsc

N = 10000
E = 320000
DI = 128
DH = 256
NC = 2
NS = 16
LANES = 16
HD = DI // NC
CHUNK = 128
EPS = E // NS
CPW = EPS // CHUNK
TAIL = EPS - CPW * CHUNK
EPSD = EPS // NC
CPWD = EPSD // CHUNK
TAILD = EPSD - CPWD * CHUNK
N_PAD = 10112
RPS = N_PAD // NS

_MESH = dict(core_axis_name="c", subcore_axis_name="s")


def _zero_stripe(zb, sh, ro):
  off = 0
  for sz in (CHUNK, CHUNK, CHUNK, CHUNK, RPS - 4 * CHUNK):
    pltpu.sync_copy(zb.at[pl.ds(0, sz)], sh.at[pl.ds(ro + off, sz)])
    off += sz


def _seg_sum_body(table, srcw, dstw, part, src_v, dst_v,
                  rows0, rows1, rows2, rows3, zb, acc_sh,
                  gs0, gs1, gs2, gs3, ss0, ss1, ss2, ss3):
  rows = (rows0, rows1, rows2, rows3)
  gsem = (gs0, gs1, gs2, gs3)
  ssem = (ss0, ss1, ss2, ss3)

  c = lax.axis_index("c")
  s = lax.axis_index("s")
  ro = s * RPS

  def zrow(i, carry):
    for j in range(HD // LANES):
      zb[i, 16 * j:16 * j + 16] = jnp.zeros((LANES,), jnp.float32)
    return carry
  lax.fori_loop(0, CHUNK, zrow, 0)

  _zero_stripe(zb, acc_sh, ro)

  pltpu.sync_copy(srcw.at[pl.ds(s * EPS, EPS)], src_v)
  pltpu.sync_copy(dstw.at[pl.ds(s * EPS, EPS)], dst_v)

  plsc.subcore_barrier()

  def gather(ci, b):
    pltpu.async_copy(table.at[c].at[src_v.at[pl.ds(ci * CHUNK, CHUNK)]],
                     rows[b], gsem[b])

  def gather_wait(b):
    pltpu.make_async_copy(table.at[c].at[src_v.at[pl.ds(0, CHUNK)]],
                          rows[b], gsem[b]).wait()

  def scatter(ci, b):
    pltpu.async_copy(rows[b], acc_sh.at[dst_v.at[pl.ds(ci * CHUNK, CHUNK)]],
                     ssem[b], add=True)

  def scatter_wait(b):
    pltpu.make_async_copy(rows[b], acc_sh.at[dst_v.at[pl.ds(0, CHUNK)]],
                          ssem[b]).wait()

  gather(0, 0)
  gather(1, 1)

  def step(k, carry):
    for b in range(4):
      ci = 4 * k + b

      @pl.when(ci >= 2)
      def _(b=b):
        scatter_wait((b + 2) % 4)

      @pl.when(ci + 2 < CPW)
      def _(b=b, ci=ci):
        gather(ci + 2, (b + 2) % 4)

      gather_wait(b)
      scatter(ci, b)
    return carry
  lax.fori_loop(0, CPW // 4, step, 0)

  scatter_wait(2)
  scatter_wait(3)

  tsrc = table.at[c].at[src_v.at[pl.ds(CPW * CHUNK, TAIL)]]
  trow = rows0.at[pl.ds(0, TAIL)]
  tdst = acc_sh.at[dst_v.at[pl.ds(CPW * CHUNK, TAIL)]]
  pltpu.async_copy(tsrc, trow, gs0)
  pltpu.make_async_copy(tsrc, trow, gs0).wait()
  pltpu.async_copy(trow, tdst, ss0, add=True)
  pltpu.make_async_copy(trow, tdst, ss0).wait()

  plsc.subcore_barrier()

  pltpu.sync_copy(acc_sh.at[pl.ds(ro, RPS)],
                  part.at[pl.ds(ro, RPS), pl.ds(c * HD, HD)])


def _make_seg_sum():
  scratch = (
      pltpu.VMEM((EPS,), jnp.int32),
      pltpu.VMEM((EPS,), jnp.int32),
      pltpu.VMEM((CHUNK, HD), jnp.float32),
      pltpu.VMEM((CHUNK, HD), jnp.float32),
      pltpu.VMEM((CHUNK, HD), jnp.float32),
      pltpu.VMEM((CHUNK, HD), jnp.float32),
      pltpu.VMEM((CHUNK, HD), jnp.float32),
      pltpu.VMEM_SHARED((N_PAD, HD), jnp.float32),
      pltpu.SemaphoreType.DMA,
      pltpu.SemaphoreType.DMA,
      pltpu.SemaphoreType.DMA,
      pltpu.SemaphoreType.DMA,
      pltpu.SemaphoreType.DMA,
      pltpu.SemaphoreType.DMA,
      pltpu.SemaphoreType.DMA,
      pltpu.SemaphoreType.DMA,
  )
  return pl.kernel(
      _seg_sum_body,
      out_type=(jax.ShapeDtypeStruct((N_PAD, DI), jnp.float32),),
      mesh=plsc.VectorSubcoreMesh(**_MESH),
      scratch_types=scratch,
      compiler_params=pltpu.CompilerParams(use_tc_tiling_on_sc=False),
      name="seg_sum",
  )


def _deg_body(dstw, degp, dst_v, ones_v, zb16, deg_sh, dsem):
  c = lax.axis_index("c")
  s = lax.axis_index("s")
  ro = s * RPS

  def frow(i, carry):
    ones_v[i] = jnp.ones((LANES,), jnp.float32)
    zb16[i] = jnp.zeros((LANES,), jnp.float32)
    return carry
  lax.fori_loop(0, CHUNK, frow, 0)

  _zero_stripe(zb16, deg_sh, ro)

  pltpu.sync_copy(dstw.at[pl.ds(s * EPS + c * EPSD, EPSD)], dst_v)

  plsc.subcore_barrier()

  def deg_wait():
    pltpu.make_async_copy(ones_v, deg_sh.at[dst_v.at[pl.ds(0, CHUNK)]],
                          dsem).wait()

  def step(k, carry):
    pltpu.async_copy(ones_v, deg_sh.at[dst_v.at[pl.ds(k * CHUNK, CHUNK)]],
                     dsem, add=True)

    @pl.when(k >= 4)
    def _():
      deg_wait()
    return carry
  lax.fori_loop(0, CPWD, step, 0)

  for _ in range(4):
    deg_wait()

  tones = ones_v.at[pl.ds(0, TAILD)]
  tdst = deg_sh.at[dst_v.at[pl.ds(CPWD * CHUNK, TAILD)]]
  pltpu.async_copy(tones, tdst, dsem, add=True)
  pltpu.make_async_copy(tones, tdst, dsem).wait()

  plsc.subcore_barrier()

  pltpu.sync_copy(deg_sh.at[pl.ds(ro, RPS)], degp.at[c, pl.ds(ro, RPS)])


def _make_deg():
  scratch = (
      pltpu.VMEM((EPSD,), jnp.int32),
      pltpu.VMEM((CHUNK, LANES), jnp.float32),
      pltpu.VMEM((CHUNK, LANES), jnp.float32),
      pltpu.VMEM_SHARED((N_PAD, LANES), jnp.float32),
      pltpu.SemaphoreType.DMA,
  )
  return pl.kernel(
      _deg_body,
      out_type=(jax.ShapeDtypeStruct((NC, N_PAD, LANES), jnp.float32),),
      mesh=plsc.VectorSubcoreMesh(**_MESH),
      scratch_types=scratch,
      compiler_params=pltpu.CompilerParams(use_tc_tiling_on_sc=False),
      name="deg_count",
  )


_seg_sum = _make_seg_sum()
_deg_count = _make_deg()

_DN = (((1,), (1,)), ((), ()))
_RB = 1000


def _deg_inv(degp_ref, i):
  deg = (degp_ref[0, pl.ds(i * _RB, _RB), 0:1]
         + degp_ref[1, pl.ds(i * _RB, _RB), 0:1])
  return 1.0 / jnp.maximum(deg, 1.0)


def _tc1_body(part_ref, degp_ref, x_ref, wl1_ref, bl1_ref, wr1_ref,
              wl2_ref, wr2_ref, bl2_ref, p_ref, q_ref):
  i = pl.program_id(0)
  inv = _deg_inv(degp_ref, i)
  mean = part_ref[...] * inv
  h = (lax.dot_general(mean, wl1_ref[...], _DN, preferred_element_type=jnp.float32)
       + bl1_ref[...]
       + lax.dot_general(x_ref[...], wr1_ref[...], _DN,
                         preferred_element_type=jnp.float32))
  p = lax.dot_general(h, wl2_ref[...], _DN, preferred_element_type=jnp.float32)
  p_ref[0] = p[:, :HD]
  p_ref[1] = p[:, HD:]
  q_ref[...] = (lax.dot_general(h, wr2_ref[...], _DN,
                                preferred_element_type=jnp.float32)
                + bl2_ref[...])


def _tc2_body(part_ref, degp_ref, q_ref, out_ref):
  i = pl.program_id(0)
  inv = _deg_inv(degp_ref, i)
  out_ref[...] = part_ref[...] * inv + q_ref[...]


def _tc1(part, degp, x, Wl1, bl1, Wr1, Wl2, Wr2, bl2):
  grid = (N // _RB,)
  return pl.pallas_call(
      _tc1_body,
      grid=grid,
      in_specs=[
          pl.BlockSpec((_RB, DI), lambda i: (i, 0)),
          pl.BlockSpec((NC, N_PAD, LANES), lambda i: (0, 0, 0)),
          pl.BlockSpec((_RB, DI), lambda i: (i, 0)),
          pl.BlockSpec((DH, DI), lambda i: (0, 0)),
          pl.BlockSpec((1, DH), lambda i: (0, 0)),
          pl.BlockSpec((DH, DI), lambda i: (0, 0)),
          pl.BlockSpec((DI, DH), lambda i: (0, 0)),
          pl.BlockSpec((DI, DH), lambda i: (0, 0)),
          pl.BlockSpec((1, DI), lambda i: (0, 0)),
      ],
      out_specs=[
          pl.BlockSpec((NC, _RB, HD), lambda i: (0, i, 0)),
          pl.BlockSpec((_RB, DI), lambda i: (i, 0)),
      ],
      out_shape=[
          jax.ShapeDtypeStruct((NC, N, HD), jnp.float32),
          jax.ShapeDtypeStruct((N, DI), jnp.float32),
      ],
      name="sage_dense1",
  )(part, degp, x, Wl1, bl1, Wr1, Wl2, Wr2, bl2)


def _tc2(part, degp, q):
  grid = (N // _RB,)
  return pl.pallas_call(
      _tc2_body,
      grid=grid,
      in_specs=[
          pl.BlockSpec((_RB, DI), lambda i: (i, 0)),
          pl.BlockSpec((NC, N_PAD, LANES), lambda i: (0, 0, 0)),
          pl.BlockSpec((_RB, DI), lambda i: (i, 0)),
      ],
      out_specs=pl.BlockSpec((_RB, DI), lambda i: (i, 0)),
      out_shape=jax.ShapeDtypeStruct((N, DI), jnp.float32),
      name="sage_dense2",
  )(part, degp, q)


def kernel(x, edge_index, Wl1, bl1, Wr1, Wl2, bl2, Wr2):
  src = edge_index[0]
  dst = edge_index[1]
  xh = jnp.stack([x[:, :HD], x[:, HD:]], axis=0)

  (part1,) = _seg_sum(xh, src, dst)
  (degp,) = _deg_count(dst)
  p, q = _tc1(part1, degp, x, Wl1, bl1[None, :], Wr1, Wl2, Wr2, bl2[None, :])
  (part2,) = _seg_sum(p, src, dst)
  return _tc2(part2, degp, q)
